# Initial kernel scaffold; baseline (speedup 1.0000x reference)
import jax
import jax.numpy as jnp
from jax.experimental import pallas as pl


def kernel(x, edge_index, edge_attr):
    raise NotImplementedError("write your pallas kernel here")


# speedup vs baseline: 6.3353x; 6.3353x over previous
import functools

import jax
import jax.numpy as jnp
from jax import lax
from jax.experimental import pallas as pl
from jax.experimental.pallas import tpu as pltpu
from jax.experimental.pallas import tpu_sc as plsc

N = 10000
E = 320000
D = 16
N_PAD = 10240
NC = 2
NS = 16
NW = NC * NS
E_PER_W = E // NW
SCAT = 125
WIN = 5000
N_SCAT = WIN // SCAT
N_WIN = E_PER_W // WIN
ROWS_PER_S = N_PAD // NS


def _sc_partials(dst3, attr, zrow, zcnt, ones):
  mesh = plsc.VectorSubcoreMesh(
      core_axis_name="c", subcore_axis_name="s", num_cores=NC, num_subcores=NS)

  @functools.partial(
      pl.kernel,
      out_type=[
          jax.ShapeDtypeStruct((NC, N_PAD, D), jnp.float32),
          jax.ShapeDtypeStruct((NC, N_PAD, D), jnp.float32),
      ],
      mesh=mesh,
      compiler_params=pltpu.CompilerParams(use_tc_tiling_on_sc=False),
      scratch_types=[
          pltpu.VMEM_SHARED((N_PAD, D), jnp.float32),
          pltpu.VMEM_SHARED((N_PAD, D), jnp.float32),
          pltpu.VMEM((N_SCAT, SCAT), jnp.int32),
          pltpu.VMEM((WIN, D), jnp.float32),
          pltpu.VMEM((SCAT, D), jnp.float32),
      ],
  )
  def k(dst_hbm, attr_hbm, zrow_hbm, zcnt_hbm, ones_hbm,
        psum_hbm, pcnt_hbm, acc, cnt, idx_v, attr_v, ones_v):
    c = lax.axis_index("c")
    s = lax.axis_index("s")
    wid = s * NC + c
    rbase = s * ROWS_PER_S

    pltpu.sync_copy(zrow_hbm, acc.at[pl.ds(rbase, ROWS_PER_S)])
    pltpu.sync_copy(zcnt_hbm, cnt.at[pl.ds(rbase, ROWS_PER_S)])
    pltpu.sync_copy(ones_hbm, ones_v)
    plsc.subcore_barrier()

    def window(w, carry):
      ebase = pl.multiple_of(wid * E_PER_W + w * WIN, WIN)
      row = pl.multiple_of(ebase // SCAT, WIN // SCAT)
      pltpu.sync_copy(dst_hbm.at[pl.ds(row, N_SCAT)], idx_v)
      pltpu.sync_copy(attr_hbm.at[pl.ds(ebase, WIN)], attr_v)
      for j in range(N_SCAT):
        pltpu.sync_copy(attr_v.at[pl.ds(j * SCAT, SCAT)],
                        acc.at[idx_v.at[j]], add=True)
        pltpu.sync_copy(ones_v, cnt.at[idx_v.at[j]], add=True)
      return carry

    lax.fori_loop(0, N_WIN, window, 0, unroll=True)
    plsc.subcore_barrier()

    pltpu.sync_copy(acc.at[pl.ds(rbase, ROWS_PER_S)],
                    psum_hbm.at[c, pl.ds(rbase, ROWS_PER_S)])
    pltpu.sync_copy(cnt.at[pl.ds(rbase, ROWS_PER_S)],
                    pcnt_hbm.at[c, pl.ds(rbase, ROWS_PER_S)])

  return k(dst3, attr, zrow, zcnt, ones)


def _combine(psum, pcnt):
  def body(ps_ref, pc_ref, out_ref):
    sums = ps_ref[0] + ps_ref[1]
    counts = pc_ref[0] + pc_ref[1]
    out_ref[...] = (sums / jnp.clip(counts, 1.0, None))[:N]

  return pl.pallas_call(
      body,
      out_shape=jax.ShapeDtypeStruct((N, D), jnp.float32),
  )(psum, pcnt)


@jax.jit
def kernel(x, edge_index, edge_attr):
  del x
  dst = edge_index[1].astype(jnp.int32)
  dst3 = dst.reshape(E // SCAT, SCAT)
  zrow = jnp.zeros((ROWS_PER_S, D), jnp.float32)
  zcnt = jnp.zeros((ROWS_PER_S, D), jnp.float32)
  ones = jnp.ones((SCAT, D), jnp.float32)
  psum, pcnt = _sc_partials(dst3, edge_attr, zrow, zcnt, ones)
  return _combine(psum, pcnt)


# speedup vs baseline: 7.2641x; 1.1466x over previous
import functools

import jax
import jax.numpy as jnp
from jax import lax
from jax.experimental import pallas as pl
from jax.experimental.pallas import tpu as pltpu
from jax.experimental.pallas import tpu_sc as plsc

N = 10000
E = 320000
D = 16
N_PAD = 10240
NC = 2
NS = 16
NW = NC * NS
E_PER_W = E // NW
SCAT = 125
WIN = 2000
N_SCAT = WIN // SCAT
N_WIN = E_PER_W // WIN
ROWS_PER_S = N_PAD // NS

_MESH = dict(core_axis_name="c", subcore_axis_name="s",
             num_cores=NC, num_subcores=NS)


def _sc_sums(dst3, attr, zrow):
  @functools.partial(
      pl.kernel,
      out_type=jax.ShapeDtypeStruct((NC, N_PAD, D), jnp.float32),
      mesh=plsc.VectorSubcoreMesh(**_MESH),
      compiler_params=pltpu.CompilerParams(use_tc_tiling_on_sc=False),
      scratch_types=[
          pltpu.VMEM_SHARED((N_PAD, D), jnp.float32),
          pltpu.VMEM((2, N_SCAT, SCAT), jnp.int32),
          pltpu.VMEM((2, WIN, D), jnp.float32),
          pltpu.SemaphoreType.DMA,
          pltpu.SemaphoreType.DMA,
      ],
  )
  def k(dst_hbm, attr_hbm, zrow_hbm, psum_hbm, acc, idx_v, attr_v,
        sem_in, sem_sc):
    c = lax.axis_index("c")
    s = lax.axis_index("s")
    wid = s * NC + c
    rbase = s * ROWS_PER_S

    pltpu.sync_copy(zrow_hbm, acc.at[pl.ds(rbase, ROWS_PER_S)])
    plsc.subcore_barrier()

    def fire_in(w):
      b = w % 2
      ebase = pl.multiple_of(wid * E_PER_W + w * WIN, WIN)
      row = pl.multiple_of(ebase // SCAT, N_SCAT)
      return [
          pltpu.async_copy(dst_hbm.at[pl.ds(row, N_SCAT)], idx_v.at[b],
                           sem_in),
          pltpu.async_copy(attr_hbm.at[pl.ds(ebase, WIN)], attr_v.at[b],
                           sem_in),
      ]

    in_descs = {0: fire_in(0)}
    for w in range(N_WIN):
      b = w % 2
      for d in in_descs.pop(w):
        d.wait()
      if w + 1 < N_WIN:
        in_descs[w + 1] = fire_in(w + 1)

      sc_descs = [
          pltpu.async_copy(attr_v.at[b, pl.ds(j * SCAT, SCAT)],
                           acc.at[idx_v.at[b, j]], sem_sc, add=True)
          for j in range(N_SCAT)
      ]
      for d in sc_descs:
        d.wait()

    plsc.subcore_barrier()

    pltpu.sync_copy(acc.at[pl.ds(rbase, ROWS_PER_S)],
                    psum_hbm.at[c, pl.ds(rbase, ROWS_PER_S)])

  return k(dst3, attr, zrow)


def _sc_counts(dstf):
  @functools.partial(
      pl.kernel,
      out_type=jax.ShapeDtypeStruct((NW, N_PAD), jnp.float32),
      mesh=plsc.VectorSubcoreMesh(**_MESH),
      compiler_params=pltpu.CompilerParams(
          use_tc_tiling_on_sc=False, needs_layout_passes=False),
      scratch_types=[
          pltpu.VMEM((2, WIN), jnp.int32),
          pltpu.VMEM((N_PAD,), jnp.float32),
          pltpu.SemaphoreType.DMA,
      ],
  )
  def k(dstf_hbm, pcnt_hbm, idxf_v, hist, sem_in):
    c = lax.axis_index("c")
    s = lax.axis_index("s")
    wid = s * NC + c
    ones16 = jnp.ones((16,), jnp.float32)
    zeros16 = jnp.zeros((16,), jnp.float32)

    def zbody(g, carry):
      hist[pl.ds(g * 16, 16)] = zeros16
      return carry
    lax.fori_loop(0, N_PAD // 16, zbody, 0, unroll=8)

    def fire(w):
      b = w % 2
      ebase = pl.multiple_of(wid * E_PER_W + w * WIN, WIN)
      return pltpu.async_copy(dstf_hbm.at[pl.ds(ebase, WIN)], idxf_v.at[b],
                              sem_in)

    descs = {0: fire(0)}
    for w in range(N_WIN):
      b = w % 2
      descs.pop(w).wait()
      if w + 1 < N_WIN:
        descs[w + 1] = fire(w + 1)

      def hbody(g, carry, b=b):
        iv = idxf_v[b, pl.ds(g * 16, 16)]
        plsc.addupdate_scatter(hist, [iv], ones16)
        return carry
      lax.fori_loop(0, WIN // 16, hbody, 0, unroll=5)

    pltpu.sync_copy(hist, pcnt_hbm.at[wid])

  return k(dstf)


def _combine(psum, pcnt_t):
  def body(ps_ref, pc_ref, out_ref):
    sums = ps_ref[0] + ps_ref[1]
    counts = jnp.sum(pc_ref[...], axis=1, keepdims=True)
    out_ref[...] = (sums / jnp.clip(counts, 1.0, None))[:N]

  return pl.pallas_call(
      body,
      out_shape=jax.ShapeDtypeStruct((N, D), jnp.float32),
  )(psum, pcnt_t)


@jax.jit
def kernel(x, edge_index, edge_attr):
  del x
  dst = edge_index[1].astype(jnp.int32)
  dst3 = dst.reshape(E // SCAT, SCAT)
  zrow = jnp.zeros((ROWS_PER_S, D), jnp.float32)
  psum = _sc_sums(dst3, edge_attr, zrow)
  pcnt = _sc_counts(dst)
  return _combine(psum, pcnt.T)


# speedup vs baseline: 7.3083x; 1.0061x over previous
import functools

import jax
import jax.numpy as jnp
from jax import lax
from jax.experimental import pallas as pl
from jax.experimental.pallas import tpu as pltpu
from jax.experimental.pallas import tpu_sc as plsc

N = 10000
E = 320000
D = 16
N_PAD = 10240
NC = 2
NS = 16
NW = NC * NS
E_PER_W = E // NW
SCAT = 125
WIN = 2000
N_SCAT = WIN // SCAT
N_WIN = E_PER_W // WIN
ROWS_PER_S = N_PAD // NS

_MESH = dict(core_axis_name="c", subcore_axis_name="s",
             num_cores=NC, num_subcores=NS)


def _sc_sums(dst3, attr, zrow):
  @functools.partial(
      pl.kernel,
      out_type=jax.ShapeDtypeStruct((NC, N_PAD, D), jnp.float32),
      mesh=plsc.VectorSubcoreMesh(**_MESH),
      compiler_params=pltpu.CompilerParams(use_tc_tiling_on_sc=False),
      scratch_types=[
          pltpu.VMEM_SHARED((N_PAD, D), jnp.float32),
          pltpu.VMEM((2, N_SCAT, SCAT), jnp.int32),
          pltpu.VMEM((2, WIN, D), jnp.float32),
          pltpu.SemaphoreType.DMA,
          pltpu.SemaphoreType.DMA,
      ],
  )
  def k(dst_hbm, attr_hbm, zrow_hbm, psum_hbm, acc, idx_v, attr_v,
        sem_in, sem_sc):
    c = lax.axis_index("c")
    s = lax.axis_index("s")
    wid = s * NC + c
    rbase = s * ROWS_PER_S

    pltpu.sync_copy(zrow_hbm, acc.at[pl.ds(rbase, ROWS_PER_S)])
    plsc.subcore_barrier()

    def fire_in(w):
      b = w % 2
      ebase = pl.multiple_of(wid * E_PER_W + w * WIN, WIN)
      row = pl.multiple_of(ebase // SCAT, N_SCAT)
      return [
          pltpu.async_copy(dst_hbm.at[pl.ds(row, N_SCAT)], idx_v.at[b],
                           sem_in),
          pltpu.async_copy(attr_hbm.at[pl.ds(ebase, WIN)], attr_v.at[b],
                           sem_in),
      ]

    in_descs = {0: fire_in(0)}
    for w in range(N_WIN):
      b = w % 2
      for d in in_descs.pop(w):
        d.wait()
      if w + 1 < N_WIN:
        in_descs[w + 1] = fire_in(w + 1)

      sc_descs = [
          pltpu.async_copy(attr_v.at[b, pl.ds(j * SCAT, SCAT)],
                           acc.at[idx_v.at[b, j]], sem_sc, add=True)
          for j in range(N_SCAT)
      ]
      for d in sc_descs:
        d.wait()

    plsc.subcore_barrier()

    pltpu.sync_copy(acc.at[pl.ds(rbase, ROWS_PER_S)],
                    psum_hbm.at[c, pl.ds(rbase, ROWS_PER_S)])

  return k(dst3, attr, zrow)


def _sc_counts(dstf):
  @functools.partial(
      pl.kernel,
      out_type=jax.ShapeDtypeStruct((NW, N_PAD), jnp.float32),
      mesh=plsc.VectorSubcoreMesh(**_MESH),
      compiler_params=pltpu.CompilerParams(
          use_tc_tiling_on_sc=False, needs_layout_passes=False),
      scratch_types=[
          pltpu.VMEM((2, WIN), jnp.int32),
          pltpu.VMEM((N_PAD,), jnp.float32),
          pltpu.SemaphoreType.DMA,
      ],
  )
  def k(dstf_hbm, pcnt_hbm, idxf_v, hist, sem_in):
    c = lax.axis_index("c")
    s = lax.axis_index("s")
    wid = s * NC + c
    ones16 = jnp.ones((16,), jnp.float32)
    zeros16 = jnp.zeros((16,), jnp.float32)

    def zbody(g, carry):
      hist[pl.ds(g * 16, 16)] = zeros16
      return carry
    lax.fori_loop(0, N_PAD // 16, zbody, 0, unroll=8)

    def fire(w):
      b = w % 2
      ebase = pl.multiple_of(wid * E_PER_W + w * WIN, WIN)
      return pltpu.async_copy(dstf_hbm.at[pl.ds(ebase, WIN)], idxf_v.at[b],
                              sem_in)

    descs = {0: fire(0)}
    for w in range(N_WIN):
      b = w % 2
      descs.pop(w).wait()
      if w + 1 < N_WIN:
        descs[w + 1] = fire(w + 1)

      def hbody(g, carry, b=b):
        iv = idxf_v[b, pl.ds(g * 16, 16)]
        plsc.addupdate_scatter(hist, [iv], ones16)
        return carry
      lax.fori_loop(0, WIN // 16, hbody, 0, unroll=5)

    pltpu.sync_copy(hist, pcnt_hbm.at[wid])

  return k(dstf)


def _combine(psum, pcnt):
  def body(ps_ref, pc_ref, out_ref):
    sums = ps_ref[0] + ps_ref[1]
    ones = jnp.ones((NW, 1), jnp.float32)
    counts = jax.lax.dot_general(
        pc_ref[...], ones, (((0,), (0,)), ((), ())),
        preferred_element_type=jnp.float32)
    out_ref[...] = (sums / jnp.clip(counts, 1.0, None))[:N]

  return pl.pallas_call(
      body,
      out_shape=jax.ShapeDtypeStruct((N, D), jnp.float32),
  )(psum, pcnt)


@jax.jit
def kernel(x, edge_index, edge_attr):
  del x
  dst = edge_index[1].astype(jnp.int32)
  dst3 = dst.reshape(E // SCAT, SCAT)
  zrow = jnp.zeros((ROWS_PER_S, D), jnp.float32)
  psum = _sc_sums(dst3, edge_attr, zrow)
  pcnt = _sc_counts(dst)
  return _combine(psum, pcnt)


# speedup vs baseline: 7.3573x; 1.0067x over previous
import functools

import jax
import jax.numpy as jnp
from jax import lax
from jax.experimental import pallas as pl
from jax.experimental.pallas import tpu as pltpu
from jax.experimental.pallas import tpu_sc as plsc

N = 10000
E = 320000
D = 16
N_PAD = 10240
NC = 2
NS = 16
NW = NC * NS
E_PER_W = E // NW
WIN = 2000
CHUNKS = [(j * 128, 128) for j in range(15)] + [(1920, 80)]
N_WIN = E_PER_W // WIN
ROWS_PER_S = N_PAD // NS

_MESH = dict(core_axis_name="c", subcore_axis_name="s",
             num_cores=NC, num_subcores=NS)


def _sc_sums(dstf, attr, zrow):
  @functools.partial(
      pl.kernel,
      out_type=jax.ShapeDtypeStruct((NC, N_PAD, D), jnp.float32),
      mesh=plsc.VectorSubcoreMesh(**_MESH),
      compiler_params=pltpu.CompilerParams(use_tc_tiling_on_sc=False),
      scratch_types=[
          pltpu.VMEM_SHARED((N_PAD, D), jnp.float32),
          pltpu.VMEM((2, WIN), jnp.int32),
          pltpu.VMEM((2, WIN, D), jnp.float32),
          pltpu.SemaphoreType.DMA,
          pltpu.SemaphoreType.DMA,
      ],
  )
  def k(dst_hbm, attr_hbm, zrow_hbm, psum_hbm, acc, idx_v, attr_v,
        sem_in, sem_sc):
    c = lax.axis_index("c")
    s = lax.axis_index("s")
    wid = s * NC + c
    rbase = s * ROWS_PER_S

    pltpu.sync_copy(zrow_hbm, acc.at[pl.ds(rbase, ROWS_PER_S)])
    plsc.subcore_barrier()

    def fire_in(w):
      b = w % 2
      ebase = pl.multiple_of(wid * E_PER_W + w * WIN, WIN)
      return [
          pltpu.async_copy(dst_hbm.at[pl.ds(ebase, WIN)], idx_v.at[b],
                           sem_in),
          pltpu.async_copy(attr_hbm.at[pl.ds(ebase, WIN)], attr_v.at[b],
                           sem_in),
      ]

    in_descs = {0: fire_in(0)}
    for w in range(N_WIN):
      b = w % 2
      for d in in_descs.pop(w):
        d.wait()
      if w + 1 < N_WIN:
        in_descs[w + 1] = fire_in(w + 1)

      sc_descs = [
          pltpu.async_copy(attr_v.at[b, pl.ds(off, ln)],
                           acc.at[idx_v.at[b, pl.ds(off, ln)]],
                           sem_sc, add=True)
          for off, ln in CHUNKS
      ]
      for d in sc_descs:
        d.wait()

    plsc.subcore_barrier()

    pltpu.sync_copy(acc.at[pl.ds(rbase, ROWS_PER_S)],
                    psum_hbm.at[c, pl.ds(rbase, ROWS_PER_S)])

  return k(dstf, attr, zrow)


def _sc_counts(dstf):
  @functools.partial(
      pl.kernel,
      out_type=jax.ShapeDtypeStruct((NW, N_PAD), jnp.float32),
      mesh=plsc.VectorSubcoreMesh(**_MESH),
      compiler_params=pltpu.CompilerParams(
          use_tc_tiling_on_sc=False, needs_layout_passes=False),
      scratch_types=[
          pltpu.VMEM((2, WIN), jnp.int32),
          pltpu.VMEM((N_PAD,), jnp.float32),
          pltpu.SemaphoreType.DMA,
      ],
  )
  def k(dstf_hbm, pcnt_hbm, idxf_v, hist, sem_in):
    c = lax.axis_index("c")
    s = lax.axis_index("s")
    wid = s * NC + c
    ones16 = jnp.ones((16,), jnp.float32)
    zeros16 = jnp.zeros((16,), jnp.float32)

    def zbody(g, carry):
      hist[pl.ds(g * 16, 16)] = zeros16
      return carry
    lax.fori_loop(0, N_PAD // 16, zbody, 0, unroll=8)

    def fire(w):
      b = w % 2
      ebase = pl.multiple_of(wid * E_PER_W + w * WIN, WIN)
      return pltpu.async_copy(dstf_hbm.at[pl.ds(ebase, WIN)], idxf_v.at[b],
                              sem_in)

    descs = {0: fire(0)}
    for w in range(N_WIN):
      b = w % 2
      descs.pop(w).wait()
      if w + 1 < N_WIN:
        descs[w + 1] = fire(w + 1)

      def hbody(g, carry, b=b):
        iv = idxf_v[b, pl.ds(g * 16, 16)]
        plsc.addupdate_scatter(hist, [iv], ones16)
        return carry
      lax.fori_loop(0, WIN // 16, hbody, 0, unroll=5)

    pltpu.sync_copy(hist, pcnt_hbm.at[wid])

  return k(dstf)


def _combine(psum, pcnt):
  def body(ps_ref, pc_ref, out_ref):
    sums = ps_ref[0] + ps_ref[1]
    ones = jnp.ones((NW, 1), jnp.float32)
    counts = jax.lax.dot_general(
        pc_ref[...], ones, (((0,), (0,)), ((), ())),
        preferred_element_type=jnp.float32)
    out_ref[...] = (sums / jnp.clip(counts, 1.0, None))[:N]

  return pl.pallas_call(
      body,
      out_shape=jax.ShapeDtypeStruct((N, D), jnp.float32),
  )(psum, pcnt)


@jax.jit
def kernel(x, edge_index, edge_attr):
  del x
  dst = edge_index[1].astype(jnp.int32)
  zrow = jnp.zeros((ROWS_PER_S, D), jnp.float32)
  psum = _sc_sums(dst, edge_attr, zrow)
  pcnt = _sc_counts(dst)
  return _combine(psum, pcnt)
